# baseline (speedup 1.0000x reference)
import jax
import jax.numpy as jnp
from jax import lax
from jax.experimental import pallas as pl
from jax.experimental.pallas import tpu as pltpu
from jax.experimental.pallas import tpu_sc as plsc

DIM = 1024
INTER = 2816
E = 8
TOP_K = 2
T = 256
NPAIR = T * TOP_K
TT = 64
NTT = T // TT
XS_ROWS = 640
IT = 1408
NI = INTER // IT
NW = 32
CHUNK = NPAIR // NW


def _router_kernel(x_ref, wr_ref, br_ref, dest_ref, w_ref, offs_ref):
    x = x_ref[...]
    logits = jnp.dot(x, wr_ref[...], preferred_element_type=jnp.float32)
    logits = logits + br_ref[...]
    m = jnp.max(logits, axis=1, keepdims=True)
    ex = jnp.exp(logits - m)
    probs = ex / jnp.sum(ex, axis=1, keepdims=True)

    lane8 = lax.broadcasted_iota(jnp.int32, (T, E), 1)
    p1 = jnp.max(probs, axis=1, keepdims=True)
    i1 = jnp.min(jnp.where(probs == p1, lane8, E), axis=1, keepdims=True)
    oh1 = (lane8 == i1)
    probs2 = jnp.where(oh1, -1.0, probs)
    p2 = jnp.max(probs2, axis=1, keepdims=True)
    i2 = jnp.min(jnp.where(probs2 == p2, lane8, E), axis=1, keepdims=True)
    oh2 = (lane8 == i2)
    psum = p1 + p2
    w_ref[...] = jnp.concatenate([p1 / psum, p2 / psum], axis=0)

    a8 = jnp.concatenate([oh1, oh2], axis=0).astype(jnp.float32)
    a16 = jnp.concatenate([a8, jnp.zeros_like(a8)], axis=1)

    r = lax.broadcasted_iota(jnp.int32, (NPAIR, NPAIR), 0)
    c = lax.broadcasted_iota(jnp.int32, (NPAIR, NPAIR), 1)
    ltri = (r > c).astype(jnp.float32)
    pos = jnp.dot(ltri, a16, preferred_element_type=jnp.float32)
    counts = jnp.sum(a16, axis=0, keepdims=True)
    aligned8 = jnp.floor((counts + 7.0) / 8.0)
    r16 = lax.broadcasted_iota(jnp.int32, (16, 16), 0)
    c16 = lax.broadcasted_iota(jnp.int32, (16, 16), 1)
    u16 = (r16 < c16).astype(jnp.float32)
    offs8 = jnp.dot(aligned8, u16, preferred_element_type=jnp.float32)

    dest = jnp.sum((pos + offs8 * 8.0) * a16, axis=1, keepdims=True)
    dest_ref[...] = dest.astype(jnp.int32)

    c32 = lax.broadcasted_iota(jnp.int32, (16, 32), 1)
    r32 = lax.broadcasted_iota(jnp.int32, (16, 32), 0)
    place = (r32 == c32).astype(jnp.float32)
    place_hi = (r32 + 16 == c32).astype(jnp.float32)
    base = jnp.dot(offs8, place, preferred_element_type=jnp.float32)
    shifted = jnp.dot(counts, place_hi, preferred_element_type=jnp.float32)
    offs_ref[...] = (base + shifted).astype(jnp.int32)


def _sc_dispatch(x_hbm, dest_hbm, w_hbm, offs_hbm,
                 xs_hbm, ps_hbm,
                 tok_ref, dest_ref, wv_ref, offs_ref,
                 rows_ref, rowsps_ref, zrow_ref, zps_ref, sem1, sem2):
    cid = lax.axis_index("c")
    sid = lax.axis_index("s")
    wid = sid * 2 + cid
    toff = (wid % 16) * CHUNK

    iota = lax.iota(jnp.int32, 16)

    pltpu.sync_copy(dest_hbm.at[pl.ds(wid * CHUNK, CHUNK)], dest_ref)
    pltpu.sync_copy(w_hbm.at[pl.ds(wid * CHUNK, CHUNK)], wv_ref)
    pltpu.sync_copy(offs_hbm, offs_ref)
    tok_ref[...] = jnp.broadcast_to(toff, (16,)) + iota
    wv = wv_ref[...]

    zf = jnp.zeros((16,), jnp.float32)
    for r in range(CHUNK):
        for c in range(T // 16):
            rowsps_ref[r, pl.ds(c * 16, 16)] = zf
    for c in range(T // 16):
        @pl.when(toff == c * 16)
        def _():
            for r in range(CHUNK):
                rowsps_ref[r, pl.ds(c * 16, 16)] = jnp.where(iota == r, wv, zf)

    pltpu.async_copy(x_hbm.at[tok_ref], rows_ref, sem1).wait()
    pltpu.async_copy(rows_ref, xs_hbm.at[dest_ref], sem1).wait()
    pltpu.async_copy(rowsps_ref, ps_hbm.at[dest_ref], sem2).wait()

    for c in range(DIM // 16):
        zrow_ref[0, pl.ds(c * 16, 16)] = zf
    for c in range(T // 16):
        zps_ref[0, pl.ds(c * 16, 16)] = zf

    offv = offs_ref[pl.ds(0, 16)]
    cntv = offs_ref[pl.ds(16, 16)]

    for t in range(E):
        @pl.when(wid == t)
        def _():
            row0 = offv[t] * 8 + cntv[t]
            gap = offv[t + 1] * 8 - row0
            for g in range(7):
                @pl.when(g < gap)
                def _():
                    pltpu.sync_copy(zrow_ref, xs_hbm.at[pl.ds(row0 + g, 1), :])
                    pltpu.sync_copy(zps_ref, ps_hbm.at[pl.ds(row0 + g, 1), :])

    @pl.when((wid >= E) & (wid < 16))
    def _():
        total = offv[E] * 8
        for g in range(16):
            row = total + (wid - E) * 16 + g

            @pl.when(row < XS_ROWS)
            def _():
                pltpu.sync_copy(zrow_ref, xs_hbm.at[pl.ds(row, 1), :])
                pltpu.sync_copy(zps_ref, ps_hbm.at[pl.ds(row, 1), :])


def _ffn_kernel(offs_ref, xs_ref, ps_ref, wg_ref, bg_ref, wu_ref, bu_ref,
                wd_ref, bd_ref, out_ref, acc_ref):
    e = pl.program_id(0)
    i = pl.program_id(1)
    off = offs_ref[e] * 8
    n = offs_ref[16 + e]

    @pl.when((e == 0) & (i == 0))
    def _():
        out_ref[...] = jnp.zeros_like(out_ref)

    for tt in range(NTT):
        @pl.when(tt * TT < n)
        def _():
            xg = xs_ref[pl.ds(off + tt * TT, TT), :]
            g = jnp.dot(xg, wg_ref[0], preferred_element_type=jnp.float32)
            g = g + bg_ref[0]
            u = jnp.dot(xg, wu_ref[0], preferred_element_type=jnp.float32)
            u = u + bu_ref[0]
            h = (g * jax.nn.sigmoid(g)) * u
            d = jnp.dot(h, wd_ref[0], preferred_element_type=jnp.float32)

            @pl.when(i == 0)
            def _():
                acc_ref[tt * TT:(tt + 1) * TT, :] = d

            @pl.when(i > 0)
            def _():
                acc_ref[tt * TT:(tt + 1) * TT, :] += d

    @pl.when(i == NI - 1)
    def _():
        for tt in range(NTT):
            @pl.when(tt * TT < n)
            def _():
                rem = n - tt * TT
                riota = lax.broadcasted_iota(jnp.int32, (TT, 1), 0)
                mask = (riota < rem).astype(jnp.float32)
                psm = ps_ref[pl.ds(off + tt * TT, TT), :] * mask
                y = acc_ref[tt * TT:(tt + 1) * TT, :] + bd_ref[0]
                out_ref[...] += lax.dot_general(
                    psm, y, (((0,), (0,)), ((), ())),
                    preferred_element_type=jnp.float32)


@jax.jit
def kernel(hidden_states, Wg, bg, Wu, bu, Wd, bd, Wr, br):
    batch, seq, dim = hidden_states.shape
    x = hidden_states.reshape(-1, dim)

    dest, w, offs = pl.pallas_call(
        _router_kernel,
        out_shape=(
            jax.ShapeDtypeStruct((NPAIR, 1), jnp.int32),
            jax.ShapeDtypeStruct((NPAIR, 1), jnp.float32),
            jax.ShapeDtypeStruct((1, 32), jnp.int32),
        ),
    )(x, Wr, br.reshape(1, E))

    mesh = plsc.VectorSubcoreMesh(core_axis_name="c", subcore_axis_name="s")
    xs, ps = pl.kernel(
        _sc_dispatch,
        mesh=mesh,
        out_type=(
            jax.ShapeDtypeStruct((XS_ROWS, DIM), jnp.float32),
            jax.ShapeDtypeStruct((XS_ROWS, T), jnp.float32),
        ),
        scratch_types=[
            pltpu.VMEM((CHUNK,), jnp.int32),
            pltpu.VMEM((CHUNK,), jnp.int32),
            pltpu.VMEM((CHUNK,), jnp.float32),
            pltpu.VMEM((32,), jnp.int32),
            pltpu.VMEM((CHUNK, DIM), jnp.float32),
            pltpu.VMEM((CHUNK, T), jnp.float32),
            pltpu.VMEM((1, DIM), jnp.float32),
            pltpu.VMEM((1, T), jnp.float32),
            pltpu.SemaphoreType.DMA,
            pltpu.SemaphoreType.DMA,
        ],
    )(x, dest.reshape(NPAIR), w.reshape(NPAIR), offs.reshape(32))

    out = pl.pallas_call(
        _ffn_kernel,
        grid_spec=pltpu.PrefetchScalarGridSpec(
            num_scalar_prefetch=1,
            grid=(E, NI),
            in_specs=[
                pl.BlockSpec((XS_ROWS, DIM), lambda e, i, offs: (0, 0)),
                pl.BlockSpec((XS_ROWS, T), lambda e, i, offs: (0, 0)),
                pl.BlockSpec((1, DIM, IT), lambda e, i, offs: (e, 0, i)),
                pl.BlockSpec((1, 1, IT), lambda e, i, offs: (e, 0, i)),
                pl.BlockSpec((1, DIM, IT), lambda e, i, offs: (e, 0, i)),
                pl.BlockSpec((1, 1, IT), lambda e, i, offs: (e, 0, i)),
                pl.BlockSpec((1, IT, DIM), lambda e, i, offs: (e, i, 0)),
                pl.BlockSpec((1, 1, DIM), lambda e, i, offs: (e, 0, 0)),
            ],
            out_specs=pl.BlockSpec((T, DIM), lambda e, i, offs: (0, 0)),
            scratch_shapes=[pltpu.VMEM((T, DIM), jnp.float32)],
        ),
        out_shape=jax.ShapeDtypeStruct((T, DIM), jnp.float32),
        compiler_params=pltpu.CompilerParams(
            dimension_semantics=("arbitrary", "arbitrary"),
            vmem_limit_bytes=60 * 1024 * 1024,
        ),
    )(offs.reshape(32), xs, ps, Wg, bg.reshape(E, 1, INTER), Wu,
      bu.reshape(E, 1, INTER), Wd, bd.reshape(E, 1, DIM))

    return out.reshape(batch, seq, dim)


# speedup vs baseline: 1.0406x; 1.0406x over previous
import jax
import jax.numpy as jnp
from jax import lax
from jax.experimental import pallas as pl
from jax.experimental.pallas import tpu as pltpu
from jax.experimental.pallas import tpu_sc as plsc

DIM = 1024
INTER = 2816
E = 8
TOP_K = 2
T = 256
NPAIR = T * TOP_K
TT = 64
NTT = T // TT
XS_ROWS = 640
IT = 1408
NI = INTER // IT
NW = 32
CHUNK = NPAIR // NW


def _router_kernel(x_ref, wr_ref, br_ref, dest_ref, w_ref, offs_ref):
    x = x_ref[...]
    logits = jnp.dot(x, wr_ref[...], preferred_element_type=jnp.float32)
    logits = logits + br_ref[...]
    m = jnp.max(logits, axis=1, keepdims=True)
    ex = jnp.exp(logits - m)
    probs = ex / jnp.sum(ex, axis=1, keepdims=True)

    lane8 = lax.broadcasted_iota(jnp.int32, (T, E), 1)
    p1 = jnp.max(probs, axis=1, keepdims=True)
    i1 = jnp.min(jnp.where(probs == p1, lane8, E), axis=1, keepdims=True)
    oh1 = (lane8 == i1)
    probs2 = jnp.where(oh1, -1.0, probs)
    p2 = jnp.max(probs2, axis=1, keepdims=True)
    i2 = jnp.min(jnp.where(probs2 == p2, lane8, E), axis=1, keepdims=True)
    oh2 = (lane8 == i2)
    psum = p1 + p2
    w_ref[...] = jnp.concatenate([p1 / psum, p2 / psum], axis=0)

    a8 = jnp.concatenate([oh1, oh2], axis=0).astype(jnp.float32)
    a16 = jnp.concatenate([a8, jnp.zeros_like(a8)], axis=1)

    r = lax.broadcasted_iota(jnp.int32, (NPAIR, NPAIR), 0)
    c = lax.broadcasted_iota(jnp.int32, (NPAIR, NPAIR), 1)
    ltri = (r > c).astype(jnp.float32)
    pos = jnp.dot(ltri, a16, preferred_element_type=jnp.float32)
    counts = jnp.sum(a16, axis=0, keepdims=True)
    aligned8 = jnp.floor((counts + 7.0) / 8.0)
    r16 = lax.broadcasted_iota(jnp.int32, (16, 16), 0)
    c16 = lax.broadcasted_iota(jnp.int32, (16, 16), 1)
    u16 = (r16 < c16).astype(jnp.float32)
    offs8 = jnp.dot(aligned8, u16, preferred_element_type=jnp.float32)

    dest = jnp.sum((pos + offs8 * 8.0) * a16, axis=1, keepdims=True)
    dest_ref[...] = dest.astype(jnp.int32)

    c32 = lax.broadcasted_iota(jnp.int32, (16, 32), 1)
    r32 = lax.broadcasted_iota(jnp.int32, (16, 32), 0)
    place = (r32 == c32).astype(jnp.float32)
    place_hi = (r32 + 16 == c32).astype(jnp.float32)
    base = jnp.dot(offs8, place, preferred_element_type=jnp.float32)
    shifted = jnp.dot(counts, place_hi, preferred_element_type=jnp.float32)
    offs_ref[...] = (base + shifted).astype(jnp.int32)


def _sc_dispatch(x_hbm, dest_hbm, w_hbm,
                 xs_hbm, ps_hbm,
                 tok_ref, dest_ref, wv_ref,
                 rows_ref, rowsps_ref, sem1, sem2):
    cid = lax.axis_index("c")
    sid = lax.axis_index("s")
    wid = sid * 2 + cid
    toff = (wid % 16) * CHUNK

    iota = lax.iota(jnp.int32, 16)

    pltpu.sync_copy(dest_hbm.at[pl.ds(wid * CHUNK, CHUNK)], dest_ref)
    pltpu.sync_copy(w_hbm.at[pl.ds(wid * CHUNK, CHUNK)], wv_ref)
    tok_ref[...] = jnp.broadcast_to(toff, (16,)) + iota
    wv = wv_ref[...]

    zf = jnp.zeros((16,), jnp.float32)
    for r in range(CHUNK):
        for c in range(T // 16):
            rowsps_ref[r, pl.ds(c * 16, 16)] = zf
    for c in range(T // 16):
        @pl.when(toff == c * 16)
        def _():
            for r in range(CHUNK):
                rowsps_ref[r, pl.ds(c * 16, 16)] = jnp.where(iota == r, wv, zf)

    gather = pltpu.async_copy(x_hbm.at[tok_ref], rows_ref, sem1)
    ps_scatter = pltpu.async_copy(rowsps_ref, ps_hbm.at[dest_ref], sem2)
    gather.wait()
    pltpu.async_copy(rows_ref, xs_hbm.at[dest_ref], sem1).wait()
    ps_scatter.wait()


def _ffn_kernel(offs_ref, xs_ref, ps_ref, wg_ref, bg_ref, wu_ref, bu_ref,
                wd_ref, bd_ref, out_ref, acc_ref):
    e = pl.program_id(0)
    i = pl.program_id(1)
    off = offs_ref[e] * 8
    n = offs_ref[16 + e]

    @pl.when((e == 0) & (i == 0))
    def _():
        out_ref[...] = jnp.zeros_like(out_ref)

    for tt in range(NTT):
        @pl.when(tt * TT < n)
        def _():
            xg = xs_ref[pl.ds(off + tt * TT, TT), :]
            g = jnp.dot(xg, wg_ref[0], preferred_element_type=jnp.float32)
            g = g + bg_ref[0]
            u = jnp.dot(xg, wu_ref[0], preferred_element_type=jnp.float32)
            u = u + bu_ref[0]
            h = (g * jax.nn.sigmoid(g)) * u
            d = jnp.dot(h, wd_ref[0], preferred_element_type=jnp.float32)

            @pl.when(i == 0)
            def _():
                acc_ref[tt * TT:(tt + 1) * TT, :] = d

            @pl.when(i > 0)
            def _():
                acc_ref[tt * TT:(tt + 1) * TT, :] += d

    @pl.when(i == NI - 1)
    def _():
        for tt in range(NTT):
            @pl.when(tt * TT < n)
            def _():
                rem = n - tt * TT
                riota = lax.broadcasted_iota(jnp.int32, (TT, 1), 0)
                rmask = riota < rem
                psm = jnp.where(rmask, ps_ref[pl.ds(off + tt * TT, TT), :],
                                0.0)
                y = jnp.where(rmask,
                              acc_ref[tt * TT:(tt + 1) * TT, :] + bd_ref[0],
                              0.0)
                out_ref[...] += lax.dot_general(
                    psm, y, (((0,), (0,)), ((), ())),
                    preferred_element_type=jnp.float32)


@jax.jit
def kernel(hidden_states, Wg, bg, Wu, bu, Wd, bd, Wr, br):
    batch, seq, dim = hidden_states.shape
    x = hidden_states.reshape(-1, dim)

    dest, w, offs = pl.pallas_call(
        _router_kernel,
        out_shape=(
            jax.ShapeDtypeStruct((NPAIR, 1), jnp.int32),
            jax.ShapeDtypeStruct((NPAIR, 1), jnp.float32),
            jax.ShapeDtypeStruct((1, 32), jnp.int32),
        ),
    )(x, Wr, br.reshape(1, E))

    mesh = plsc.VectorSubcoreMesh(core_axis_name="c", subcore_axis_name="s")
    xs, ps = pl.kernel(
        _sc_dispatch,
        mesh=mesh,
        out_type=(
            jax.ShapeDtypeStruct((XS_ROWS, DIM), jnp.float32),
            jax.ShapeDtypeStruct((XS_ROWS, T), jnp.float32),
        ),
        scratch_types=[
            pltpu.VMEM((CHUNK,), jnp.int32),
            pltpu.VMEM((CHUNK,), jnp.int32),
            pltpu.VMEM((CHUNK,), jnp.float32),
            pltpu.VMEM((CHUNK, DIM), jnp.float32),
            pltpu.VMEM((CHUNK, T), jnp.float32),
            pltpu.SemaphoreType.DMA,
            pltpu.SemaphoreType.DMA,
        ],
    )(x, dest.reshape(NPAIR), w.reshape(NPAIR))

    out = pl.pallas_call(
        _ffn_kernel,
        grid_spec=pltpu.PrefetchScalarGridSpec(
            num_scalar_prefetch=1,
            grid=(E, NI),
            in_specs=[
                pl.BlockSpec((XS_ROWS, DIM), lambda e, i, offs: (0, 0)),
                pl.BlockSpec((XS_ROWS, T), lambda e, i, offs: (0, 0)),
                pl.BlockSpec((1, DIM, IT), lambda e, i, offs: (e, 0, i)),
                pl.BlockSpec((1, 1, IT), lambda e, i, offs: (e, 0, i)),
                pl.BlockSpec((1, DIM, IT), lambda e, i, offs: (e, 0, i)),
                pl.BlockSpec((1, 1, IT), lambda e, i, offs: (e, 0, i)),
                pl.BlockSpec((1, IT, DIM), lambda e, i, offs: (e, i, 0)),
                pl.BlockSpec((1, 1, DIM), lambda e, i, offs: (e, 0, 0)),
            ],
            out_specs=pl.BlockSpec((T, DIM), lambda e, i, offs: (0, 0)),
            scratch_shapes=[pltpu.VMEM((T, DIM), jnp.float32)],
        ),
        out_shape=jax.ShapeDtypeStruct((T, DIM), jnp.float32),
        compiler_params=pltpu.CompilerParams(
            dimension_semantics=("arbitrary", "arbitrary"),
            vmem_limit_bytes=60 * 1024 * 1024,
        ),
    )(offs.reshape(32), xs, ps, Wg, bg.reshape(E, 1, INTER), Wu,
      bu.reshape(E, 1, INTER), Wd, bd.reshape(E, 1, DIM))

    return out.reshape(batch, seq, dim)


# speedup vs baseline: 1.0439x; 1.0032x over previous
import jax
import jax.numpy as jnp
from jax import lax
from jax.experimental import pallas as pl
from jax.experimental.pallas import tpu as pltpu
from jax.experimental.pallas import tpu_sc as plsc

DIM = 1024
INTER = 2816
E = 8
TOP_K = 2
T = 256
NPAIR = T * TOP_K
TT = 64
NTT = T // TT
XS_ROWS = 640
IT = 1408
NI = INTER // IT
NW = 32
CHUNK = NPAIR // NW


def _router_kernel(x_ref, wr_ref, br_ref, dest_ref, w_ref, offs_ref):
    x = x_ref[...]
    logits = jnp.dot(x, wr_ref[...], preferred_element_type=jnp.float32)
    logits = logits + br_ref[...]
    m = jnp.max(logits, axis=1, keepdims=True)
    ex = jnp.exp(logits - m)
    probs = ex / jnp.sum(ex, axis=1, keepdims=True)

    lane8 = lax.broadcasted_iota(jnp.int32, (T, E), 1)
    p1 = jnp.max(probs, axis=1, keepdims=True)
    i1 = jnp.min(jnp.where(probs == p1, lane8, E), axis=1, keepdims=True)
    oh1 = (lane8 == i1)
    probs2 = jnp.where(oh1, -1.0, probs)
    p2 = jnp.max(probs2, axis=1, keepdims=True)
    i2 = jnp.min(jnp.where(probs2 == p2, lane8, E), axis=1, keepdims=True)
    oh2 = (lane8 == i2)
    psum = p1 + p2
    w_ref[...] = jnp.concatenate([p1 / psum, p2 / psum], axis=0)

    a8 = jnp.concatenate([oh1, oh2], axis=0).astype(jnp.float32)
    a16 = jnp.concatenate([a8, jnp.zeros_like(a8)], axis=1)

    r = lax.broadcasted_iota(jnp.int32, (NPAIR, NPAIR), 0)
    c = lax.broadcasted_iota(jnp.int32, (NPAIR, NPAIR), 1)
    ltri = (r > c).astype(jnp.float32)
    pos = jnp.dot(ltri, a16, preferred_element_type=jnp.float32)
    counts = jnp.sum(a16, axis=0, keepdims=True)
    aligned8 = jnp.floor((counts + 7.0) / 8.0)
    r16 = lax.broadcasted_iota(jnp.int32, (16, 16), 0)
    c16 = lax.broadcasted_iota(jnp.int32, (16, 16), 1)
    u16 = (r16 < c16).astype(jnp.float32)
    offs8 = jnp.dot(aligned8, u16, preferred_element_type=jnp.float32)

    dest = jnp.sum((pos + offs8 * 8.0) * a16, axis=1, keepdims=True)
    dest_ref[...] = dest.astype(jnp.int32)

    c32 = lax.broadcasted_iota(jnp.int32, (16, 32), 1)
    r32 = lax.broadcasted_iota(jnp.int32, (16, 32), 0)
    place = (r32 == c32).astype(jnp.float32)
    place_hi = (r32 + 16 == c32).astype(jnp.float32)
    base = jnp.dot(offs8, place, preferred_element_type=jnp.float32)
    shifted = jnp.dot(counts, place_hi, preferred_element_type=jnp.float32)
    offs_ref[...] = (base + shifted).astype(jnp.int32)


def _sc_dispatch(x_hbm, dest_hbm, w_hbm,
                 xs_hbm, ps_hbm,
                 tok_ref, dest_ref, wv_ref,
                 rows_ref, rowsps_ref, sem1, sem2):
    cid = lax.axis_index("c")
    sid = lax.axis_index("s")
    wid = sid * 2 + cid
    toff = (wid % 16) * CHUNK

    iota = lax.iota(jnp.int32, 16)

    pltpu.sync_copy(dest_hbm.at[pl.ds(wid * CHUNK, CHUNK)], dest_ref)
    pltpu.sync_copy(w_hbm.at[pl.ds(wid * CHUNK, CHUNK)], wv_ref)
    tok_ref[...] = jnp.broadcast_to(toff, (16,)) + iota
    wv = wv_ref[...]

    zf = jnp.zeros((16,), jnp.float32)
    for r in range(CHUNK):
        for c in range(T // 16):
            rowsps_ref[r, pl.ds(c * 16, 16)] = zf
    for c in range(T // 16):
        @pl.when(toff == c * 16)
        def _():
            for r in range(CHUNK):
                rowsps_ref[r, pl.ds(c * 16, 16)] = jnp.where(iota == r, wv, zf)

    load = pltpu.async_copy(x_hbm.at[pl.ds(toff, CHUNK), :], rows_ref, sem1)
    ps_scatter = pltpu.async_copy(rowsps_ref, ps_hbm.at[dest_ref], sem2)
    load.wait()
    pltpu.async_copy(rows_ref, xs_hbm.at[dest_ref], sem1).wait()
    ps_scatter.wait()


def _ffn_kernel(offs_ref, xs_ref, ps_ref, wg_ref, bg_ref, wu_ref, bu_ref,
                wd_ref, bd_ref, out_ref, acc_ref):
    e = pl.program_id(0)
    i = pl.program_id(1)
    off = offs_ref[e] * 8
    n = offs_ref[16 + e]

    @pl.when((e == 0) & (i == 0))
    def _():
        out_ref[...] = jnp.zeros_like(out_ref)

    for tt in range(NTT):
        @pl.when(tt * TT < n)
        def _():
            xg = xs_ref[pl.ds(off + tt * TT, TT), :]
            g = jnp.dot(xg, wg_ref[0], preferred_element_type=jnp.float32)
            g = g + bg_ref[0]
            u = jnp.dot(xg, wu_ref[0], preferred_element_type=jnp.float32)
            u = u + bu_ref[0]
            h = (g * jax.nn.sigmoid(g)) * u
            d = jnp.dot(h, wd_ref[0], preferred_element_type=jnp.float32)

            @pl.when(i == 0)
            def _():
                acc_ref[tt * TT:(tt + 1) * TT, :] = d

            @pl.when(i > 0)
            def _():
                acc_ref[tt * TT:(tt + 1) * TT, :] += d

    @pl.when(i == NI - 1)
    def _():
        for tt in range(NTT):
            @pl.when(tt * TT < n)
            def _():
                rem = n - tt * TT
                riota = lax.broadcasted_iota(jnp.int32, (TT, 1), 0)
                rmask = riota < rem
                psm = jnp.where(rmask, ps_ref[pl.ds(off + tt * TT, TT), :],
                                0.0)
                y = jnp.where(rmask,
                              acc_ref[tt * TT:(tt + 1) * TT, :] + bd_ref[0],
                              0.0)
                out_ref[...] += lax.dot_general(
                    psm, y, (((0,), (0,)), ((), ())),
                    preferred_element_type=jnp.float32)


@jax.jit
def kernel(hidden_states, Wg, bg, Wu, bu, Wd, bd, Wr, br):
    batch, seq, dim = hidden_states.shape
    x = hidden_states.reshape(-1, dim)

    dest, w, offs = pl.pallas_call(
        _router_kernel,
        out_shape=(
            jax.ShapeDtypeStruct((NPAIR, 1), jnp.int32),
            jax.ShapeDtypeStruct((NPAIR, 1), jnp.float32),
            jax.ShapeDtypeStruct((1, 32), jnp.int32),
        ),
    )(x, Wr, br.reshape(1, E))

    mesh = plsc.VectorSubcoreMesh(core_axis_name="c", subcore_axis_name="s")
    xs, ps = pl.kernel(
        _sc_dispatch,
        mesh=mesh,
        out_type=(
            jax.ShapeDtypeStruct((XS_ROWS, DIM), jnp.float32),
            jax.ShapeDtypeStruct((XS_ROWS, T), jnp.float32),
        ),
        scratch_types=[
            pltpu.VMEM((CHUNK,), jnp.int32),
            pltpu.VMEM((CHUNK,), jnp.int32),
            pltpu.VMEM((CHUNK,), jnp.float32),
            pltpu.VMEM((CHUNK, DIM), jnp.float32),
            pltpu.VMEM((CHUNK, T), jnp.float32),
            pltpu.SemaphoreType.DMA,
            pltpu.SemaphoreType.DMA,
        ],
    )(x, dest.reshape(NPAIR), w.reshape(NPAIR))

    out = pl.pallas_call(
        _ffn_kernel,
        grid_spec=pltpu.PrefetchScalarGridSpec(
            num_scalar_prefetch=1,
            grid=(E, NI),
            in_specs=[
                pl.BlockSpec((XS_ROWS, DIM), lambda e, i, offs: (0, 0)),
                pl.BlockSpec((XS_ROWS, T), lambda e, i, offs: (0, 0)),
                pl.BlockSpec((1, DIM, IT), lambda e, i, offs: (e, 0, i)),
                pl.BlockSpec((1, 1, IT), lambda e, i, offs: (e, 0, i)),
                pl.BlockSpec((1, DIM, IT), lambda e, i, offs: (e, 0, i)),
                pl.BlockSpec((1, 1, IT), lambda e, i, offs: (e, 0, i)),
                pl.BlockSpec((1, IT, DIM), lambda e, i, offs: (e, i, 0)),
                pl.BlockSpec((1, 1, DIM), lambda e, i, offs: (e, 0, 0)),
            ],
            out_specs=pl.BlockSpec((T, DIM), lambda e, i, offs: (0, 0)),
            scratch_shapes=[pltpu.VMEM((T, DIM), jnp.float32)],
        ),
        out_shape=jax.ShapeDtypeStruct((T, DIM), jnp.float32),
        compiler_params=pltpu.CompilerParams(
            dimension_semantics=("arbitrary", "arbitrary"),
            vmem_limit_bytes=60 * 1024 * 1024,
        ),
    )(offs.reshape(32), xs, ps, Wg, bg.reshape(E, 1, INTER), Wu,
      bu.reshape(E, 1, INTER), Wd, bd.reshape(E, 1, DIM))

    return out.reshape(batch, seq, dim)
